# baseline (speedup 1.0000x reference)
import functools

import jax
import jax.numpy as jnp
from jax import lax
from jax.experimental import pallas as pl
from jax.experimental.pallas import tpu as pltpu
from jax.experimental.pallas import tpu_sc as plsc

K_NN = 3
QB = 512
SC_CORES = 2
SC_SUBCORES = 16
SC_CHUNK = 128


def _knn_body(n1, x2_ref, x2b_ref, x1b_ref, tn_ref, idx_ref, w_ref):
    b = pl.program_id(0)
    q = x2_ref[0]
    qb = x2b_ref[0]
    tb = x1b_ref[0]
    tnh = tn_ref[0]
    qnh = jnp.sum(q * q, axis=1, keepdims=True) * 0.5
    e = lax.dot_general(qb, tb, (((1,), (0,)), ((), ())),
                        preferred_element_type=jnp.float32)
    d2 = (qnh + tnh) - e
    iota = lax.broadcasted_iota(jnp.int32, d2.shape, 1).astype(jnp.float32)
    idxs, dists = [], []
    for k in range(K_NN):
        m = jnp.min(d2, axis=1, keepdims=True)
        sel = jnp.where(d2 == m, iota, jnp.float32(n1))
        ik = jnp.min(sel, axis=1, keepdims=True)
        if k + 1 < K_NN:
            d2 = jnp.where(sel == ik, jnp.float32(jnp.inf), d2)
        dists.append(jnp.sqrt(jnp.maximum(m + m, 1e-12)))
        idxs.append(ik)
    w = [1.0 / (d + 1e-8) for d in dists]
    ws = w[0] + w[1] + w[2]
    w = [x / ws for x in w]
    idx_f = jnp.concatenate(idxs, axis=1)
    idx_ref[0] = idx_f.astype(jnp.int32) + b * n1
    w_ref[0] = jnp.concatenate(w, axis=1)


def _knn(xyz1, xyz2_t):
    B, _, N1 = xyz1.shape
    N2 = xyz2_t.shape[1]
    x2b = xyz2_t.astype(jnp.bfloat16)
    x1b = xyz1.astype(jnp.bfloat16)
    tn = jnp.sum(xyz1 * xyz1, axis=1, keepdims=True) * 0.5
    grid = (B, N2 // QB)
    return pl.pallas_call(
        functools.partial(_knn_body, N1),
        grid=grid,
        in_specs=[
            pl.BlockSpec((1, QB, 3), lambda b, i: (b, i, 0)),
            pl.BlockSpec((1, QB, 3), lambda b, i: (b, i, 0)),
            pl.BlockSpec((1, 3, N1), lambda b, i: (b, 0, 0)),
            pl.BlockSpec((1, 1, N1), lambda b, i: (b, 0, 0)),
        ],
        out_specs=[
            pl.BlockSpec((1, QB, K_NN), lambda b, i: (b, i, 0)),
            pl.BlockSpec((1, QB, K_NN), lambda b, i: (b, i, 0)),
        ],
        out_shape=[
            jax.ShapeDtypeStruct((B, N2, K_NN), jnp.int32),
            jax.ShapeDtypeStruct((B, N2, K_NN), jnp.float32),
        ],
    )(xyz2_t, x2b, x1b, tn)


def _sc_gather(idx, table):
    n_vec, _ = idx.shape
    rows_total = n_vec * SC_CHUNK
    c_pad = table.shape[1]
    nw = SC_CORES * SC_SUBCORES
    per_w = n_vec // nw
    mesh = plsc.VectorSubcoreMesh(core_axis_name="c", subcore_axis_name="s")

    @functools.partial(
        pl.kernel,
        out_type=jax.ShapeDtypeStruct((rows_total, c_pad), jnp.float32),
        mesh=mesh,
        scratch_types=[
            pltpu.VMEM((per_w, SC_CHUNK), jnp.int32),
            pltpu.VMEM((SC_CHUNK, c_pad), jnp.float32),
            pltpu.VMEM((SC_CHUNK, c_pad), jnp.float32),
            pltpu.SemaphoreType.DMA,
            pltpu.SemaphoreType.DMA,
        ],
    )
    def body(idx_hbm, table_hbm, out_hbm, idx_v, rows0, rows1, sem0, sem1):
        wid = lax.axis_index("s") * SC_CORES + lax.axis_index("c")
        vbase = wid * per_w
        pltpu.sync_copy(idx_hbm.at[pl.ds(vbase, per_w)], idx_v)
        bufs = (rows0, rows1)
        sems = (sem0, sem1)
        handles = [None, None]
        handles[0] = pltpu.async_copy(table_hbm.at[idx_v.at[0]], rows0, sem0)
        for j in range(per_w):
            cur = j % 2
            handles[cur].wait()
            if j + 1 < per_w:
                nxt = (j + 1) % 2
                handles[nxt] = pltpu.async_copy(
                    table_hbm.at[idx_v.at[j + 1]], bufs[nxt], sems[nxt])
            pltpu.sync_copy(bufs[cur],
                            out_hbm.at[pl.ds((vbase + j) * SC_CHUNK, SC_CHUNK)])

    return body(idx, table)


def _dot(a, b):
    return lax.dot_general(a, b, (((1,), (0,)), ((), ())),
                           preferred_element_type=jnp.float32,
                           precision=lax.Precision.HIGHEST)


def _group_stats(s, ss, group_size, n_elems):
    cc = s.shape[1]
    gi = lax.broadcasted_iota(jnp.int32, (cc, cc), 0) // group_size
    gj = lax.broadcasted_iota(jnp.int32, (cc, cc), 1) // group_size
    G = (gi == gj).astype(jnp.float32)
    mean = _dot(s, G) / n_elems
    ex2 = _dot(ss, G) / n_elems
    return mean, ex2 - mean * mean


def _conv1_body(g_ref, w3_ref, f2_ref, w0a_ref, w0bp_ref, b0_ref,
                h1_ref, part_ref):
    g = g_ref[0]
    w3 = w3_ref[0]
    f2 = f2_ref[0]
    span = g.shape[1] // K_NN
    facc = lax.dot_general(f2, w0a_ref[...], (((0,), (0,)), ((), ())),
                           preferred_element_type=jnp.float32,
                           precision=lax.Precision.HIGHEST)
    acc = facc + b0_ref[...]
    for k in range(K_NN):
        wk = w3[:, k:k + 1]
        gk = g[:, k * span:(k + 1) * span]
        acc = acc + _dot(gk * wk, w0bp_ref[...])
    h1_ref[0] = acc
    s = jnp.sum(acc, axis=0, keepdims=True)
    ss = jnp.sum(acc * acc, axis=0, keepdims=True)
    part_ref[0, 0] = jnp.concatenate([s, ss], axis=0)


def _conv1(g, w3, feat2, w0a, w0bp, b0):
    B, N2, kc = g.shape
    co = w0a.shape[1]
    c2 = feat2.shape[1]
    qb = 2048
    nq = N2 // qb
    full = lambda shape: pl.BlockSpec(shape, lambda b, i: tuple(0 for _ in shape))
    return pl.pallas_call(
        _conv1_body,
        grid=(B, nq),
        in_specs=[
            pl.BlockSpec((1, qb, kc), lambda b, i: (b, i, 0)),
            pl.BlockSpec((1, qb, K_NN), lambda b, i: (b, i, 0)),
            pl.BlockSpec((1, c2, qb), lambda b, i: (b, 0, i)),
            full(w0a.shape), full(w0bp.shape), full(b0.shape),
        ],
        out_specs=[
            pl.BlockSpec((1, qb, co), lambda b, i: (b, i, 0)),
            pl.BlockSpec((1, 1, 2, co), lambda b, i: (b, i, 0, 0)),
        ],
        out_shape=[
            jax.ShapeDtypeStruct((B, N2, co), jnp.float32),
            jax.ShapeDtypeStruct((B, nq, 2, co), jnp.float32),
        ],
    )(g, w3, feat2, w0a, w0bp, b0)


def _group_stats_col(s, ss, group_size, n_elems):
    cc = s.shape[0]
    gi = lax.broadcasted_iota(jnp.int32, (cc, cc), 0) // group_size
    gj = lax.broadcasted_iota(jnp.int32, (cc, cc), 1) // group_size
    G = (gi == gj).astype(jnp.float32)
    mean = _dot(G, s) / n_elems
    ex2 = _dot(G, ss) / n_elems
    return mean, ex2 - mean * mean


def _gn_mlp_body(h1_ref, part_ref, gs0_ref, gb0_ref, w1_ref, b1_ref,
                 gs1_ref, gb1_ref, out_ref):
    h1 = h1_ref[0]
    parts = part_ref[0]
    n2 = h1.shape[0]
    s = jnp.sum(parts[:, 0, :], axis=0, keepdims=True)
    ss = jnp.sum(parts[:, 1, :], axis=0, keepdims=True)
    mean, var = _group_stats(s, ss, 16, n2 * 16)
    h = (h1 - mean) * lax.rsqrt(var + 1e-5) * gs0_ref[...] + gb0_ref[...]
    h = jnp.where(h >= 0, h, 0.1 * h)
    h2 = lax.dot_general(w1_ref[...], h, (((1,), (1,)), ((), ())),
                         preferred_element_type=jnp.float32,
                         precision=lax.Precision.HIGHEST) + b1_ref[...]
    s2 = jnp.sum(h2, axis=1, keepdims=True)
    ss2 = jnp.sum(h2 * h2, axis=1, keepdims=True)
    mean2, var2 = _group_stats_col(s2, ss2, 16, n2 * 16)
    h2 = (h2 - mean2) * lax.rsqrt(var2 + 1e-5) * gs1_ref[...] + gb1_ref[...]
    out_ref[0] = jnp.where(h2 >= 0, h2, 0.1 * h2)


def _gn_mlp(h1, parts, gs0, gb0, w1, b1, gs1, gb1):
    B, N2, co = h1.shape
    nq = parts.shape[1]
    full = lambda shape: pl.BlockSpec(shape, lambda b: tuple(0 for _ in shape))
    return pl.pallas_call(
        _gn_mlp_body,
        grid=(B,),
        in_specs=[
            pl.BlockSpec((1, N2, co), lambda b: (b, 0, 0)),
            pl.BlockSpec((1, nq, 2, co), lambda b: (b, 0, 0, 0)),
            full(gs0.shape), full(gb0.shape), full(w1.shape),
            full(b1.shape), full(gs1.shape), full(gb1.shape),
        ],
        out_specs=pl.BlockSpec((1, co, N2), lambda b: (b, 0, 0)),
        out_shape=jax.ShapeDtypeStruct((B, co, N2), jnp.float32),
    )(h1, parts, gs0, gb0, w1, b1, gs1, gb1)


def kernel(xyz1, xyz2, feat1, feat2, W0, b0, gs0, gb0, W1, b1, gs1, gb1):
    B, _, N1 = xyz1.shape
    N2 = xyz2.shape[2]
    C1 = feat1.shape[1]
    C2 = feat2.shape[1]

    xyz2_t = jnp.transpose(xyz2, (0, 2, 1))
    idx3, w3 = _knn(xyz1, xyz2_t)

    table = jnp.transpose(feat1, (0, 2, 1))
    table = jnp.concatenate(
        [table, jnp.zeros_like(table)], axis=-1).reshape(B * N1, 2 * C1)
    idx_flat = idx3.reshape(B * N2 * K_NN // SC_CHUNK, SC_CHUNK)
    g = _sc_gather(idx_flat, table)
    g = g.reshape(B, N2, K_NN * 2 * C1)

    w0a = W0[:, :C2].T
    w0b = W0[:, C2:].T
    w0bp = jnp.concatenate([w0b, jnp.zeros_like(w0b)], axis=0)
    h1, parts = _conv1(g, w3, feat2, w0a, w0bp, b0.reshape(1, -1))
    return _gn_mlp(h1, parts,
                   gs0.reshape(1, -1), gb0.reshape(1, -1),
                   W1, b1.reshape(-1, 1), gs1.reshape(-1, 1),
                   gb1.reshape(-1, 1))


# speedup vs baseline: 1.0787x; 1.0787x over previous
import functools

import jax
import jax.numpy as jnp
from jax import lax
from jax.experimental import pallas as pl
from jax.experimental.pallas import tpu as pltpu
from jax.experimental.pallas import tpu_sc as plsc

K_NN = 3
QB = 512
SC_CORES = 2
SC_SUBCORES = 16
SC_CHUNK = 128


def _knn_body(n1, x2_ref, x2b_ref, x1b_ref, tn_ref, idx_ref, w_ref):
    b = pl.program_id(0)
    q = x2_ref[0]
    qb = x2b_ref[0]
    tb = x1b_ref[0]
    tnh = tn_ref[0]
    qnh = jnp.sum(q * q, axis=1, keepdims=True) * 0.5
    e = lax.dot_general(qb, tb, (((1,), (0,)), ((), ())),
                        preferred_element_type=jnp.float32)
    d2 = (qnh + tnh) - e
    iota = lax.broadcasted_iota(jnp.int32, d2.shape, 1).astype(jnp.float32)
    idxs, dists = [], []
    for k in range(K_NN):
        m = jnp.min(d2, axis=1, keepdims=True)
        sel = jnp.where(d2 == m, iota, jnp.float32(n1))
        ik = jnp.min(sel, axis=1, keepdims=True)
        if k + 1 < K_NN:
            d2 = jnp.where(sel == ik, jnp.float32(jnp.inf), d2)
        dists.append(jnp.sqrt(jnp.maximum(m + m, 1e-12)))
        idxs.append(ik)
    w = [1.0 / (d + 1e-8) for d in dists]
    ws = w[0] + w[1] + w[2]
    w = [x / ws for x in w]
    idx_f = jnp.concatenate(idxs, axis=1)
    idx_ref[0] = idx_f.astype(jnp.int32) + b * n1
    w_ref[0] = jnp.concatenate(w, axis=1)


def _knn(xyz1, xyz2_t):
    B, _, N1 = xyz1.shape
    N2 = xyz2_t.shape[1]
    x2b = xyz2_t.astype(jnp.bfloat16)
    x1b = xyz1.astype(jnp.bfloat16)
    tn = jnp.sum(xyz1 * xyz1, axis=1, keepdims=True) * 0.5
    grid = (B, N2 // QB)
    return pl.pallas_call(
        functools.partial(_knn_body, N1),
        grid=grid,
        in_specs=[
            pl.BlockSpec((1, QB, 3), lambda b, i: (b, i, 0)),
            pl.BlockSpec((1, QB, 3), lambda b, i: (b, i, 0)),
            pl.BlockSpec((1, 3, N1), lambda b, i: (b, 0, 0)),
            pl.BlockSpec((1, 1, N1), lambda b, i: (b, 0, 0)),
        ],
        out_specs=[
            pl.BlockSpec((1, QB, K_NN), lambda b, i: (b, i, 0)),
            pl.BlockSpec((1, QB, K_NN), lambda b, i: (b, i, 0)),
        ],
        out_shape=[
            jax.ShapeDtypeStruct((B, N2, K_NN), jnp.int32),
            jax.ShapeDtypeStruct((B, N2, K_NN), jnp.float32),
        ],
    )(xyz2_t, x2b, x1b, tn)


def _sc_interp(idx, w, table, c_out):
    n_vec, _ = idx.shape
    rows_total = n_vec * SC_CHUNK
    nq_total = rows_total // K_NN
    c_pad = table.shape[1]
    nw = SC_CORES * SC_SUBCORES
    per_w = n_vec // nw
    nch = per_w // K_NN
    mesh = plsc.VectorSubcoreMesh(core_axis_name="c", subcore_axis_name="s")

    q_per_w = nq_total // nw

    @functools.partial(
        pl.kernel,
        out_type=jax.ShapeDtypeStruct((nq_total, c_out), jnp.float32),
        mesh=mesh,
        scratch_types=[
            pltpu.VMEM((per_w, SC_CHUNK), jnp.int32),
            pltpu.VMEM((K_NN, q_per_w), jnp.float32),
            pltpu.VMEM((K_NN * SC_CHUNK, c_pad), jnp.float32),
            pltpu.VMEM((SC_CHUNK, c_out), jnp.float32),
            pltpu.SemaphoreType.DMA,
        ],
    )
    def body(idx_hbm, w_hbm, table_hbm, out_hbm, idx_v, w_vm, rg, outb, sem):
        wid = lax.axis_index("s") * SC_CORES + lax.axis_index("c")
        vbase = wid * per_w
        pltpu.sync_copy(idx_hbm.at[pl.ds(vbase, per_w)], idx_v)
        pltpu.sync_copy(
            w_hbm.at[pl.ds(0, K_NN), pl.ds(wid * q_per_w, q_per_w)], w_vm)
        for c in range(nch):
            handles = [
                pltpu.async_copy(table_hbm.at[idx_v.at[K_NN * c + j]],
                                 rg.at[pl.ds(j * SC_CHUNK, SC_CHUNK)], sem)
                for j in range(K_NN)
            ]
            for h in handles:
                h.wait()

            def block16(t, carry):
                qbase = 16 * t
                wof = c * SC_CHUNK + qbase
                wv = [w_vm[j, pl.ds(wof, 16)] for j in range(K_NN)]
                for i in range(16):
                    w0, w1, w2 = wv[0][i], wv[1][i], wv[2][i]
                    rb = K_NN * (qbase + i)
                    for v in range(c_out // 16):
                        cs = pl.ds(16 * v, 16)
                        outb[qbase + i, cs] = (w0 * rg[rb + 0, cs]
                                               + w1 * rg[rb + 1, cs]
                                               + w2 * rg[rb + 2, cs])
                return carry

            lax.fori_loop(0, SC_CHUNK // 16, block16, 0)
            pltpu.sync_copy(
                outb,
                out_hbm.at[pl.ds(wid * q_per_w + c * SC_CHUNK, SC_CHUNK)])

    return body(idx, w, table)


def _dot(a, b):
    return lax.dot_general(a, b, (((1,), (0,)), ((), ())),
                           preferred_element_type=jnp.float32,
                           precision=lax.Precision.HIGHEST)


def _group_stats(s, ss, group_size, n_elems):
    cc = s.shape[1]
    gi = lax.broadcasted_iota(jnp.int32, (cc, cc), 0) // group_size
    gj = lax.broadcasted_iota(jnp.int32, (cc, cc), 1) // group_size
    G = (gi == gj).astype(jnp.float32)
    mean = _dot(s, G) / n_elems
    ex2 = _dot(ss, G) / n_elems
    return mean, ex2 - mean * mean


def _group_stats_col(s, ss, group_size, n_elems):
    cc = s.shape[0]
    gi = lax.broadcasted_iota(jnp.int32, (cc, cc), 0) // group_size
    gj = lax.broadcasted_iota(jnp.int32, (cc, cc), 1) // group_size
    G = (gi == gj).astype(jnp.float32)
    mean = _dot(G, s) / n_elems
    ex2 = _dot(G, ss) / n_elems
    return mean, ex2 - mean * mean


def _mlp_body(it_ref, f2_ref, w0a_ref, w0b_ref, b0_ref, gs0_ref, gb0_ref,
              w1_ref, b1_ref, gs1_ref, gb1_ref, out_ref):
    it = it_ref[0]
    f2 = f2_ref[0]
    n2 = it.shape[0]
    facc = lax.dot_general(f2, w0a_ref[...], (((0,), (0,)), ((), ())),
                           preferred_element_type=jnp.float32,
                           precision=lax.Precision.HIGHEST)
    h = facc + _dot(it, w0b_ref[...]) + b0_ref[...]
    s = jnp.sum(h, axis=0, keepdims=True)
    ss = jnp.sum(h * h, axis=0, keepdims=True)
    mean, var = _group_stats(s, ss, 16, n2 * 16)
    h = (h - mean) * lax.rsqrt(var + 1e-5) * gs0_ref[...] + gb0_ref[...]
    h = jnp.where(h >= 0, h, 0.1 * h)
    h2 = lax.dot_general(w1_ref[...], h, (((1,), (1,)), ((), ())),
                         preferred_element_type=jnp.float32,
                         precision=lax.Precision.HIGHEST) + b1_ref[...]
    s2 = jnp.sum(h2, axis=1, keepdims=True)
    ss2 = jnp.sum(h2 * h2, axis=1, keepdims=True)
    mean2, var2 = _group_stats_col(s2, ss2, 16, n2 * 16)
    h2 = (h2 - mean2) * lax.rsqrt(var2 + 1e-5) * gs1_ref[...] + gb1_ref[...]
    out_ref[0] = jnp.where(h2 >= 0, h2, 0.1 * h2)


def _mlp(interp, feat2, w0a, w0b, b0, gs0, gb0, w1, b1, gs1, gb1):
    B, N2, co = interp.shape
    c2 = feat2.shape[1]
    full = lambda shape: pl.BlockSpec(shape, lambda b: tuple(0 for _ in shape))
    return pl.pallas_call(
        _mlp_body,
        grid=(B,),
        in_specs=[
            pl.BlockSpec((1, N2, co), lambda b: (b, 0, 0)),
            pl.BlockSpec((1, c2, N2), lambda b: (b, 0, 0)),
            full(w0a.shape), full(w0b.shape), full(b0.shape),
            full(gs0.shape), full(gb0.shape), full(w1.shape),
            full(b1.shape), full(gs1.shape), full(gb1.shape),
        ],
        out_specs=pl.BlockSpec((1, co, N2), lambda b: (b, 0, 0)),
        out_shape=jax.ShapeDtypeStruct((B, co, N2), jnp.float32),
    )(interp, feat2, w0a, w0b, b0, gs0, gb0, w1, b1, gs1, gb1)


def kernel(xyz1, xyz2, feat1, feat2, W0, b0, gs0, gb0, W1, b1, gs1, gb1):
    B, _, N1 = xyz1.shape
    N2 = xyz2.shape[2]
    C1 = feat1.shape[1]
    C2 = feat2.shape[1]

    xyz2_t = jnp.transpose(xyz2, (0, 2, 1))
    idx3, w3 = _knn(xyz1, xyz2_t)

    table = jnp.transpose(feat1, (0, 2, 1))
    table = jnp.concatenate(
        [table, jnp.zeros_like(table)], axis=-1).reshape(B * N1, 2 * C1)
    idx_flat = idx3.reshape(B * N2 * K_NN // SC_CHUNK, SC_CHUNK)
    w_km = w3.reshape(B * N2, K_NN).T
    interp = _sc_interp(idx_flat, w_km, table, C1)
    interp = interp.reshape(B, N2, C1)

    w0a = W0[:, :C2].T
    w0b = W0[:, C2:].T
    return _mlp(interp, feat2, w0a, w0b,
                b0.reshape(1, -1), gs0.reshape(1, -1), gb0.reshape(1, -1),
                W1, b1.reshape(-1, 1), gs1.reshape(-1, 1),
                gb1.reshape(-1, 1))


# speedup vs baseline: 1.1298x; 1.0474x over previous
import functools

import jax
import jax.numpy as jnp
from jax import lax
from jax.experimental import pallas as pl
from jax.experimental.pallas import tpu as pltpu
from jax.experimental.pallas import tpu_sc as plsc

K_NN = 3
QB = 512
SC_CORES = 2
SC_SUBCORES = 16
SC_CHUNK = 128


def _knn_body(n1, x2_ref, x2b_ref, x1b_ref, tn_ref, idx_ref, w_ref):
    b = pl.program_id(0)
    q = x2_ref[0]
    qb = x2b_ref[0]
    tb = x1b_ref[0]
    tnh = tn_ref[0]
    qnh = jnp.sum(q * q, axis=1, keepdims=True) * 0.5
    e = lax.dot_general(qb, tb, (((1,), (0,)), ((), ())),
                        preferred_element_type=jnp.float32)
    d2 = (qnh + tnh) - e
    iota = lax.broadcasted_iota(jnp.int32, d2.shape, 1).astype(jnp.float32)
    idxs, dists = [], []
    for k in range(K_NN):
        m = jnp.min(d2, axis=1, keepdims=True)
        sel = jnp.where(d2 == m, iota, jnp.float32(n1))
        ik = jnp.min(sel, axis=1, keepdims=True)
        if k + 1 < K_NN:
            d2 = jnp.where(sel == ik, jnp.float32(jnp.inf), d2)
        dists.append(jnp.sqrt(jnp.maximum(m + m, 1e-12)))
        idxs.append(ik)
    w = [1.0 / (d + 1e-8) for d in dists]
    ws = w[0] + w[1] + w[2]
    w = [x / ws for x in w]
    idx_f = jnp.concatenate(idxs, axis=1)
    idx_ref[0] = idx_f.astype(jnp.int32) + b * n1
    w_ref[0] = jnp.concatenate(w, axis=1)


def _knn(xyz1, xyz2_t):
    B, _, N1 = xyz1.shape
    N2 = xyz2_t.shape[1]
    x2b = xyz2_t.astype(jnp.bfloat16)
    x1b = xyz1.astype(jnp.bfloat16)
    tn = jnp.sum(xyz1 * xyz1, axis=1, keepdims=True) * 0.5
    grid = (B, N2 // QB)
    return pl.pallas_call(
        functools.partial(_knn_body, N1),
        grid=grid,
        in_specs=[
            pl.BlockSpec((1, QB, 3), lambda b, i: (b, i, 0)),
            pl.BlockSpec((1, QB, 3), lambda b, i: (b, i, 0)),
            pl.BlockSpec((1, 3, N1), lambda b, i: (b, 0, 0)),
            pl.BlockSpec((1, 1, N1), lambda b, i: (b, 0, 0)),
        ],
        out_specs=[
            pl.BlockSpec((1, QB, K_NN), lambda b, i: (b, i, 0)),
            pl.BlockSpec((1, QB, K_NN), lambda b, i: (b, i, 0)),
        ],
        out_shape=[
            jax.ShapeDtypeStruct((B, N2, K_NN), jnp.int32),
            jax.ShapeDtypeStruct((B, N2, K_NN), jnp.float32),
        ],
    )(xyz2_t, x2b, x1b, tn)


def _sc_interp(idx, w, table, c_out):
    n_vec, _ = idx.shape
    rows_total = n_vec * SC_CHUNK
    nq_total = rows_total // K_NN
    c_pad = table.shape[1]
    nw = SC_CORES * SC_SUBCORES
    per_w = n_vec // nw
    nch = per_w // K_NN
    mesh = plsc.VectorSubcoreMesh(core_axis_name="c", subcore_axis_name="s")

    q_per_w = nq_total // nw

    @functools.partial(
        pl.kernel,
        out_type=jax.ShapeDtypeStruct((nq_total, c_out), jnp.float32),
        mesh=mesh,
        scratch_types=[
            pltpu.VMEM((per_w, SC_CHUNK), jnp.int32),
            pltpu.VMEM((K_NN, q_per_w), jnp.float32),
            pltpu.VMEM((K_NN * SC_CHUNK, c_pad), jnp.float32),
            pltpu.VMEM((K_NN * SC_CHUNK, c_pad), jnp.float32),
            pltpu.VMEM((SC_CHUNK, c_out), jnp.float32),
            pltpu.SemaphoreType.DMA,
            pltpu.SemaphoreType.DMA,
            pltpu.SemaphoreType.DMA,
        ],
    )
    def body(idx_hbm, w_hbm, table_hbm, out_hbm, idx_v, w_vm, rg0, rg1,
             outb, sg0, sg1, sw0):
        wid = lax.axis_index("s") * SC_CORES + lax.axis_index("c")
        vbase = wid * per_w
        pltpu.sync_copy(idx_hbm.at[pl.ds(vbase, per_w)], idx_v)
        pltpu.sync_copy(
            w_hbm.at[pl.ds(0, K_NN), pl.ds(wid * q_per_w, q_per_w)], w_vm)
        rgs = (rg0, rg1)
        sgs = (sg0, sg1)

        def gather_chunk(c):
            cur = c % 2
            return [
                pltpu.async_copy(table_hbm.at[idx_v.at[K_NN * c + j]],
                                 rgs[cur].at[pl.ds(j * SC_CHUNK, SC_CHUNK)],
                                 sgs[cur])
                for j in range(K_NN)
            ]

        gh = {0: gather_chunk(0)}
        wh = {}
        for c in range(nch):
            cur = c % 2
            if c + 1 < nch:
                gh[c + 1] = gather_chunk(c + 1)
            for h in gh.pop(c):
                h.wait()
            if c - 1 in wh:
                wh.pop(c - 1).wait()
            rg = rgs[cur]

            def block16(t, carry):
                qbase = 16 * t
                wof = c * SC_CHUNK + qbase
                wv = [w_vm[j, pl.ds(wof, 16)] for j in range(K_NN)]
                for i in range(16):
                    w0, w1, w2 = wv[0][i], wv[1][i], wv[2][i]
                    rb = K_NN * (qbase + i)
                    for v in range(c_out // 16):
                        cs = pl.ds(16 * v, 16)
                        outb[qbase + i, cs] = (w0 * rg[rb + 0, cs]
                                               + w1 * rg[rb + 1, cs]
                                               + w2 * rg[rb + 2, cs])
                return carry

            lax.fori_loop(0, SC_CHUNK // 16, block16, 0)
            wh[c] = pltpu.async_copy(
                outb,
                out_hbm.at[pl.ds(wid * q_per_w + c * SC_CHUNK, SC_CHUNK)],
                sw0)
        for h in wh.values():
            h.wait()

    return body(idx, w, table)


def _dot(a, b):
    return lax.dot_general(a, b, (((1,), (0,)), ((), ())),
                           preferred_element_type=jnp.float32,
                           precision=lax.Precision.HIGHEST)


def _group_stats(s, ss, group_size, n_elems):
    cc = s.shape[1]
    gi = lax.broadcasted_iota(jnp.int32, (cc, cc), 0) // group_size
    gj = lax.broadcasted_iota(jnp.int32, (cc, cc), 1) // group_size
    G = (gi == gj).astype(jnp.float32)
    mean = _dot(s, G) / n_elems
    ex2 = _dot(ss, G) / n_elems
    return mean, ex2 - mean * mean


def _group_stats_col(s, ss, group_size, n_elems):
    cc = s.shape[0]
    gi = lax.broadcasted_iota(jnp.int32, (cc, cc), 0) // group_size
    gj = lax.broadcasted_iota(jnp.int32, (cc, cc), 1) // group_size
    G = (gi == gj).astype(jnp.float32)
    mean = _dot(G, s) / n_elems
    ex2 = _dot(G, ss) / n_elems
    return mean, ex2 - mean * mean


def _mlp_body(it_ref, f2_ref, w0a_ref, w0b_ref, b0_ref, gs0_ref, gb0_ref,
              w1_ref, b1_ref, gs1_ref, gb1_ref, out_ref):
    it = it_ref[0]
    f2 = f2_ref[0]
    n2 = it.shape[0]
    facc = lax.dot_general(f2, w0a_ref[...], (((0,), (0,)), ((), ())),
                           preferred_element_type=jnp.float32,
                           precision=lax.Precision.HIGHEST)
    h = facc + _dot(it, w0b_ref[...]) + b0_ref[...]
    s = jnp.sum(h, axis=0, keepdims=True)
    ss = jnp.sum(h * h, axis=0, keepdims=True)
    mean, var = _group_stats(s, ss, 16, n2 * 16)
    h = (h - mean) * lax.rsqrt(var + 1e-5) * gs0_ref[...] + gb0_ref[...]
    h = jnp.where(h >= 0, h, 0.1 * h)
    h2 = lax.dot_general(w1_ref[...], h, (((1,), (1,)), ((), ())),
                         preferred_element_type=jnp.float32,
                         precision=lax.Precision.HIGHEST) + b1_ref[...]
    s2 = jnp.sum(h2, axis=1, keepdims=True)
    ss2 = jnp.sum(h2 * h2, axis=1, keepdims=True)
    mean2, var2 = _group_stats_col(s2, ss2, 16, n2 * 16)
    h2 = (h2 - mean2) * lax.rsqrt(var2 + 1e-5) * gs1_ref[...] + gb1_ref[...]
    out_ref[0] = jnp.where(h2 >= 0, h2, 0.1 * h2)


def _mlp(interp, feat2, w0a, w0b, b0, gs0, gb0, w1, b1, gs1, gb1):
    B, N2, co = interp.shape
    c2 = feat2.shape[1]
    full = lambda shape: pl.BlockSpec(shape, lambda b: tuple(0 for _ in shape))
    return pl.pallas_call(
        _mlp_body,
        grid=(B,),
        in_specs=[
            pl.BlockSpec((1, N2, co), lambda b: (b, 0, 0)),
            pl.BlockSpec((1, c2, N2), lambda b: (b, 0, 0)),
            full(w0a.shape), full(w0b.shape), full(b0.shape),
            full(gs0.shape), full(gb0.shape), full(w1.shape),
            full(b1.shape), full(gs1.shape), full(gb1.shape),
        ],
        out_specs=pl.BlockSpec((1, co, N2), lambda b: (b, 0, 0)),
        out_shape=jax.ShapeDtypeStruct((B, co, N2), jnp.float32),
    )(interp, feat2, w0a, w0b, b0, gs0, gb0, w1, b1, gs1, gb1)


def kernel(xyz1, xyz2, feat1, feat2, W0, b0, gs0, gb0, W1, b1, gs1, gb1):
    B, _, N1 = xyz1.shape
    N2 = xyz2.shape[2]
    C1 = feat1.shape[1]
    C2 = feat2.shape[1]

    xyz2_t = jnp.transpose(xyz2, (0, 2, 1))
    idx3, w3 = _knn(xyz1, xyz2_t)

    table = jnp.transpose(feat1, (0, 2, 1))
    table = jnp.concatenate(
        [table, jnp.zeros_like(table)], axis=-1).reshape(B * N1, 2 * C1)
    idx_flat = idx3.reshape(B * N2 * K_NN // SC_CHUNK, SC_CHUNK)
    w_km = w3.reshape(B * N2, K_NN).T
    interp = _sc_interp(idx_flat, w_km, table, C1)
    interp = interp.reshape(B, N2, C1)

    w0a = W0[:, :C2].T
    w0b = W0[:, C2:].T
    return _mlp(interp, feat2, w0a, w0b,
                b0.reshape(1, -1), gs0.reshape(1, -1), gb0.reshape(1, -1),
                W1, b1.reshape(-1, 1), gs1.reshape(-1, 1),
                gb1.reshape(-1, 1))


# speedup vs baseline: 1.2369x; 1.0948x over previous
import functools

import jax
import jax.numpy as jnp
from jax import lax
from jax.experimental import pallas as pl
from jax.experimental.pallas import tpu as pltpu
from jax.experimental.pallas import tpu_sc as plsc

K_NN = 3
QB = 512
SC_CORES = 2
SC_SUBCORES = 16
SC_CHUNK = 128


def _knn_body(n1, x2_ref, x2b_ref, x1b_ref, tn_ref, idx_ref, w_ref):
    b = pl.program_id(0)
    q = x2_ref[0]
    qb = x2b_ref[0]
    tb = x1b_ref[0]
    tnh = tn_ref[0]
    qnh = jnp.sum(q * q, axis=1, keepdims=True) * 0.5
    e = lax.dot_general(qb, tb, (((1,), (0,)), ((), ())),
                        preferred_element_type=jnp.float32)
    d2 = (qnh + tnh) - e
    iota = lax.broadcasted_iota(jnp.int32, d2.shape, 1).astype(jnp.float32)
    idxs, dists = [], []
    for k in range(K_NN):
        m = jnp.min(d2, axis=1, keepdims=True)
        sel = jnp.where(d2 == m, iota, jnp.float32(n1))
        ik = jnp.min(sel, axis=1, keepdims=True)
        if k + 1 < K_NN:
            d2 = jnp.where(sel == ik, jnp.float32(jnp.inf), d2)
        dists.append(jnp.sqrt(jnp.maximum(m + m, 1e-12)))
        idxs.append(ik)
    w = [1.0 / (d + 1e-8) for d in dists]
    ws = w[0] + w[1] + w[2]
    w = [x / ws for x in w]
    idx_f = jnp.concatenate(idxs, axis=1)
    idx_ref[0] = idx_f.astype(jnp.int32) + b * n1
    w_ref[0] = jnp.concatenate(w, axis=1)


def _knn(xyz1, xyz2_t):
    B, _, N1 = xyz1.shape
    N2 = xyz2_t.shape[1]
    x2b = xyz2_t.astype(jnp.bfloat16)
    x1b = xyz1.astype(jnp.bfloat16)
    tn = jnp.sum(xyz1 * xyz1, axis=1, keepdims=True) * 0.5
    grid = (B, N2 // QB)
    return pl.pallas_call(
        functools.partial(_knn_body, N1),
        grid=grid,
        in_specs=[
            pl.BlockSpec((1, QB, 3), lambda b, i: (b, i, 0)),
            pl.BlockSpec((1, QB, 3), lambda b, i: (b, i, 0)),
            pl.BlockSpec((1, 3, N1), lambda b, i: (b, 0, 0)),
            pl.BlockSpec((1, 1, N1), lambda b, i: (b, 0, 0)),
        ],
        out_specs=[
            pl.BlockSpec((1, QB, K_NN), lambda b, i: (b, i, 0)),
            pl.BlockSpec((1, QB, K_NN), lambda b, i: (b, i, 0)),
        ],
        out_shape=[
            jax.ShapeDtypeStruct((B, N2, K_NN), jnp.int32),
            jax.ShapeDtypeStruct((B, N2, K_NN), jnp.float32),
        ],
    )(xyz2_t, x2b, x1b, tn)


def _sc_interp(idx, w, table, c_out):
    n_vec, _ = idx.shape
    rows_total = n_vec * SC_CHUNK
    nq_total = rows_total // K_NN
    c_pad = table.shape[1]
    nw = SC_CORES * SC_SUBCORES
    per_w = n_vec // nw
    nch = per_w // K_NN
    mesh = plsc.VectorSubcoreMesh(core_axis_name="c", subcore_axis_name="s")

    q_per_w = nq_total // nw

    @functools.partial(
        pl.kernel,
        out_type=jax.ShapeDtypeStruct((nq_total, c_out), jnp.float32),
        mesh=mesh,
        scratch_types=[
            pltpu.VMEM((per_w, SC_CHUNK), jnp.int32),
            pltpu.VMEM((K_NN, q_per_w), jnp.float32),
            pltpu.VMEM((K_NN * SC_CHUNK, c_pad), jnp.float32),
            pltpu.VMEM((K_NN * SC_CHUNK, c_pad), jnp.float32),
            pltpu.VMEM((SC_CHUNK, c_out), jnp.float32),
            pltpu.SemaphoreType.DMA,
            pltpu.SemaphoreType.DMA,
            pltpu.SemaphoreType.DMA,
        ],
    )
    def body(idx_hbm, w_hbm, table_hbm, out_hbm, idx_v, w_vm, rg0, rg1,
             outb, sg0, sg1, sw0):
        wid = lax.axis_index("s") * SC_CORES + lax.axis_index("c")
        vbase = wid * per_w
        pltpu.sync_copy(idx_hbm.at[pl.ds(vbase, per_w)], idx_v)
        pltpu.sync_copy(
            w_hbm.at[pl.ds(0, K_NN), pl.ds(wid * q_per_w, q_per_w)], w_vm)
        rgs = (rg0, rg1)
        sgs = (sg0, sg1)

        def gather_chunk(c):
            cur = c % 2
            return [
                pltpu.async_copy(table_hbm.at[idx_v.at[K_NN * c + j]],
                                 rgs[cur].at[pl.ds(j * SC_CHUNK, SC_CHUNK)],
                                 sgs[cur])
                for j in range(K_NN)
            ]

        gh = {0: gather_chunk(0)}
        wh = {}
        for c in range(nch):
            cur = c % 2
            if c + 1 < nch:
                gh[c + 1] = gather_chunk(c + 1)
            for h in gh.pop(c):
                h.wait()
            if c - 1 in wh:
                wh.pop(c - 1).wait()
            rg = rgs[cur]

            def block16(t, carry):
                qbase = 16 * t
                wof = c * SC_CHUNK + qbase
                wv = [w_vm[j, pl.ds(wof, 16)] for j in range(K_NN)]
                for i in range(16):
                    w0, w1, w2 = wv[0][i], wv[1][i], wv[2][i]
                    rb = K_NN * (qbase + i)
                    for v in range(c_out // 16):
                        cs = pl.ds(16 * v, 16)
                        outb[qbase + i, cs] = (w0 * rg[rb + 0, cs]
                                               + w1 * rg[rb + 1, cs]
                                               + w2 * rg[rb + 2, cs])
                return carry

            lax.fori_loop(0, SC_CHUNK // 16, block16, 0)
            wh[c] = pltpu.async_copy(
                outb,
                out_hbm.at[pl.ds(wid * q_per_w + c * SC_CHUNK, SC_CHUNK)],
                sw0)
        for h in wh.values():
            h.wait()

    return body(idx, w, table)


def _dot(a, b):
    return lax.dot_general(a, b, (((1,), (0,)), ((), ())),
                           preferred_element_type=jnp.float32,
                           precision=lax.Precision.HIGHEST)


def _group_stats(s, ss, group_size, n_elems):
    cc = s.shape[1]
    gi = lax.broadcasted_iota(jnp.int32, (cc, cc), 0) // group_size
    gj = lax.broadcasted_iota(jnp.int32, (cc, cc), 1) // group_size
    G = (gi == gj).astype(jnp.float32)
    mean = _dot(s, G) / n_elems
    ex2 = _dot(ss, G) / n_elems
    return mean, ex2 - mean * mean


def _group_stats_col(s, ss, group_size, n_elems):
    cc = s.shape[0]
    gi = lax.broadcasted_iota(jnp.int32, (cc, cc), 0) // group_size
    gj = lax.broadcasted_iota(jnp.int32, (cc, cc), 1) // group_size
    G = (gi == gj).astype(jnp.float32)
    mean = _dot(G, s) / n_elems
    ex2 = _dot(G, ss) / n_elems
    return mean, ex2 - mean * mean


def _mlp_body(it_ref, f2_ref, w0a_ref, w0b_ref, b0_ref, gs0_ref, gb0_ref,
              w1_ref, b1_ref, gs1_ref, gb1_ref, out_ref):
    it = it_ref[0]
    f2 = f2_ref[0]
    n2 = it.shape[0]
    h = (lax.dot_general(w0a_ref[...], f2, (((1,), (0,)), ((), ())),
                         preferred_element_type=jnp.float32,
                         precision=lax.Precision.HIGHEST)
         + lax.dot_general(w0b_ref[...], it, (((1,), (1,)), ((), ())),
                           preferred_element_type=jnp.float32,
                           precision=lax.Precision.HIGHEST)
         + b0_ref[...])
    s = jnp.sum(h, axis=1, keepdims=True)
    ss = jnp.sum(h * h, axis=1, keepdims=True)
    mean, var = _group_stats_col(s, ss, 16, n2 * 16)
    h = (h - mean) * lax.rsqrt(var + 1e-5) * gs0_ref[...] + gb0_ref[...]
    h = jnp.where(h >= 0, h, 0.1 * h)
    h2 = lax.dot_general(w1_ref[...], h, (((1,), (0,)), ((), ())),
                         preferred_element_type=jnp.float32,
                         precision=lax.Precision.HIGHEST) + b1_ref[...]
    s2 = jnp.sum(h2, axis=1, keepdims=True)
    ss2 = jnp.sum(h2 * h2, axis=1, keepdims=True)
    mean2, var2 = _group_stats_col(s2, ss2, 16, n2 * 16)
    h2 = (h2 - mean2) * lax.rsqrt(var2 + 1e-5) * gs1_ref[...] + gb1_ref[...]
    out_ref[0] = jnp.where(h2 >= 0, h2, 0.1 * h2)


def _mlp(interp, feat2, w0a, w0b, b0, gs0, gb0, w1, b1, gs1, gb1):
    B, N2, co = interp.shape
    c2 = feat2.shape[1]
    full = lambda shape: pl.BlockSpec(shape, lambda b: tuple(0 for _ in shape))
    return pl.pallas_call(
        _mlp_body,
        grid=(B,),
        in_specs=[
            pl.BlockSpec((1, N2, co), lambda b: (b, 0, 0)),
            pl.BlockSpec((1, c2, N2), lambda b: (b, 0, 0)),
            full(w0a.shape), full(w0b.shape), full(b0.shape),
            full(gs0.shape), full(gb0.shape), full(w1.shape),
            full(b1.shape), full(gs1.shape), full(gb1.shape),
        ],
        out_specs=pl.BlockSpec((1, co, N2), lambda b: (b, 0, 0)),
        out_shape=jax.ShapeDtypeStruct((B, co, N2), jnp.float32),
    )(interp, feat2, w0a, w0b, b0, gs0, gb0, w1, b1, gs1, gb1)


def kernel(xyz1, xyz2, feat1, feat2, W0, b0, gs0, gb0, W1, b1, gs1, gb1):
    B, _, N1 = xyz1.shape
    N2 = xyz2.shape[2]
    C1 = feat1.shape[1]
    C2 = feat2.shape[1]

    xyz2_t = jnp.transpose(xyz2, (0, 2, 1))
    idx3, w3 = _knn(xyz1, xyz2_t)

    table = jnp.transpose(feat1, (0, 2, 1))
    table = jnp.concatenate(
        [table, jnp.zeros_like(table)], axis=-1).reshape(B * N1, 2 * C1)
    idx_flat = idx3.reshape(B * N2 * K_NN // SC_CHUNK, SC_CHUNK)
    w_km = w3.reshape(B * N2, K_NN).T
    interp = _sc_interp(idx_flat, w_km, table, C1)
    interp = interp.reshape(B, N2, C1)

    w0a = W0[:, :C2]
    w0b = W0[:, C2:]
    return _mlp(interp, feat2, w0a, w0b,
                b0.reshape(-1, 1), gs0.reshape(-1, 1), gb0.reshape(-1, 1),
                W1, b1.reshape(-1, 1), gs1.reshape(-1, 1),
                gb1.reshape(-1, 1))


# speedup vs baseline: 1.2442x; 1.0059x over previous
import functools

import jax
import jax.numpy as jnp
from jax import lax
from jax.experimental import pallas as pl
from jax.experimental.pallas import tpu as pltpu
from jax.experimental.pallas import tpu_sc as plsc

K_NN = 3
QB = 512
SC_CORES = 2
SC_SUBCORES = 16
SC_CHUNK = 128


def _knn_body(n1, x2_ref, x2b_ref, x1b_ref, tn_ref, idx_ref, w_ref):
    b = pl.program_id(0)
    q = x2_ref[0]
    qb = x2b_ref[0]
    tb = x1b_ref[0]
    tnh = tn_ref[0]
    qnh = jnp.sum(q * q, axis=1, keepdims=True) * 0.5
    e = lax.dot_general(qb, tb, (((1,), (0,)), ((), ())),
                        preferred_element_type=jnp.float32)
    d2 = (qnh + tnh) - e
    iota = lax.broadcasted_iota(jnp.int32, d2.shape, 1).astype(jnp.float32)
    idxs, dists = [], []
    for k in range(K_NN):
        m = jnp.min(d2, axis=1, keepdims=True)
        sel = jnp.where(d2 == m, iota, jnp.float32(n1))
        ik = jnp.min(sel, axis=1, keepdims=True)
        if k + 1 < K_NN:
            d2 = jnp.where(sel == ik, jnp.float32(jnp.inf), d2)
        dists.append(jnp.sqrt(jnp.maximum(m + m, 1e-12)))
        idxs.append(ik)
    w = [1.0 / (d + 1e-8) for d in dists]
    ws = w[0] + w[1] + w[2]
    w = [x / ws for x in w]
    idx_f = jnp.concatenate(idxs, axis=1)
    idx_ref[0] = idx_f.astype(jnp.int32) + b * n1
    w_ref[0] = jnp.concatenate(w, axis=1)


def _knn(xyz1, xyz2_t):
    B, _, N1 = xyz1.shape
    N2 = xyz2_t.shape[1]
    x2b = xyz2_t.astype(jnp.bfloat16)
    x1b = xyz1.astype(jnp.bfloat16)
    tn = jnp.sum(xyz1 * xyz1, axis=1, keepdims=True) * 0.5
    grid = (B, N2 // QB)
    return pl.pallas_call(
        functools.partial(_knn_body, N1),
        grid=grid,
        in_specs=[
            pl.BlockSpec((1, QB, 3), lambda b, i: (b, i, 0)),
            pl.BlockSpec((1, QB, 3), lambda b, i: (b, i, 0)),
            pl.BlockSpec((1, 3, N1), lambda b, i: (b, 0, 0)),
            pl.BlockSpec((1, 1, N1), lambda b, i: (b, 0, 0)),
        ],
        out_specs=[
            pl.BlockSpec((1, QB, K_NN), lambda b, i: (b, i, 0)),
            pl.BlockSpec((1, QB, K_NN), lambda b, i: (b, i, 0)),
        ],
        out_shape=[
            jax.ShapeDtypeStruct((B, N2, K_NN), jnp.int32),
            jax.ShapeDtypeStruct((B, N2, K_NN), jnp.float32),
        ],
    )(xyz2_t, x2b, x1b, tn)


def _sc_interp(idx, w, table, c_out):
    n_vec = idx.shape[0] // SC_CHUNK
    rows_total = n_vec * SC_CHUNK
    nq_total = rows_total // K_NN
    c_pad = table.shape[1]
    nw = SC_CORES * SC_SUBCORES
    per_w = n_vec // nw
    nch = per_w // K_NN
    mesh = plsc.VectorSubcoreMesh(core_axis_name="c", subcore_axis_name="s")

    q_per_w = nq_total // nw

    @functools.partial(
        pl.kernel,
        out_type=jax.ShapeDtypeStruct((nq_total, c_out), jnp.float32),
        mesh=mesh,
        scratch_types=[
            pltpu.VMEM((per_w * SC_CHUNK,), jnp.int32),
            pltpu.VMEM((K_NN, q_per_w), jnp.float32),
            pltpu.VMEM((K_NN * SC_CHUNK, c_pad), jnp.float32),
            pltpu.VMEM((K_NN * SC_CHUNK, c_pad), jnp.float32),
            pltpu.VMEM((SC_CHUNK, c_out), jnp.float32),
            pltpu.SemaphoreType.DMA,
            pltpu.SemaphoreType.DMA,
            pltpu.SemaphoreType.DMA,
        ],
    )
    def body(idx_hbm, w_hbm, table_hbm, out_hbm, idx_v, w_vm, rg0, rg1,
             outb, sg0, sg1, sw0):
        wid = lax.axis_index("s") * SC_CORES + lax.axis_index("c")
        vbase = wid * per_w * SC_CHUNK
        pltpu.sync_copy(idx_hbm.at[pl.ds(vbase, per_w * SC_CHUNK)], idx_v)
        pltpu.sync_copy(
            w_hbm.at[pl.ds(0, K_NN), pl.ds(wid * q_per_w, q_per_w)], w_vm)
        rgs = (rg0, rg1)
        sgs = (sg0, sg1)

        def gather_chunk(c):
            cur = c % 2
            return [
                pltpu.async_copy(
                    table_hbm.at[idx_v.at[pl.ds((K_NN * c + j) * SC_CHUNK,
                                                SC_CHUNK)]],
                    rgs[cur].at[pl.ds(j * SC_CHUNK, SC_CHUNK)],
                    sgs[cur])
                for j in range(K_NN)
            ]

        gh = {0: gather_chunk(0)}
        wh = {}
        for c in range(nch):
            cur = c % 2
            if c + 1 < nch:
                gh[c + 1] = gather_chunk(c + 1)
            for h in gh.pop(c):
                h.wait()
            if c - 1 in wh:
                wh.pop(c - 1).wait()
            rg = rgs[cur]

            def block16(t, carry):
                qbase = 16 * t
                wof = c * SC_CHUNK + qbase
                wv = [w_vm[j, pl.ds(wof, 16)] for j in range(K_NN)]
                for i in range(16):
                    w0, w1, w2 = wv[0][i], wv[1][i], wv[2][i]
                    rb = K_NN * (qbase + i)
                    for v in range(c_out // 16):
                        cs = pl.ds(16 * v, 16)
                        outb[qbase + i, cs] = (w0 * rg[rb + 0, cs]
                                               + w1 * rg[rb + 1, cs]
                                               + w2 * rg[rb + 2, cs])
                return carry

            lax.fori_loop(0, SC_CHUNK // 16, block16, 0)
            wh[c] = pltpu.async_copy(
                outb,
                out_hbm.at[pl.ds(wid * q_per_w + c * SC_CHUNK, SC_CHUNK)],
                sw0)
        for h in wh.values():
            h.wait()

    return body(idx, w, table)


def _dot(a, b):
    return lax.dot_general(a, b, (((1,), (0,)), ((), ())),
                           preferred_element_type=jnp.float32,
                           precision=lax.Precision.HIGHEST)


def _group_stats(s, ss, group_size, n_elems):
    cc = s.shape[1]
    gi = lax.broadcasted_iota(jnp.int32, (cc, cc), 0) // group_size
    gj = lax.broadcasted_iota(jnp.int32, (cc, cc), 1) // group_size
    G = (gi == gj).astype(jnp.float32)
    mean = _dot(s, G) / n_elems
    ex2 = _dot(ss, G) / n_elems
    return mean, ex2 - mean * mean


def _group_stats_col(s, ss, group_size, n_elems):
    cc = s.shape[0]
    gi = lax.broadcasted_iota(jnp.int32, (cc, cc), 0) // group_size
    gj = lax.broadcasted_iota(jnp.int32, (cc, cc), 1) // group_size
    G = (gi == gj).astype(jnp.float32)
    mean = _dot(G, s) / n_elems
    ex2 = _dot(G, ss) / n_elems
    return mean, ex2 - mean * mean


def _mlp_split_body(it0_ref, it1_ref, f2_ref, w0a_ref, w0b_ref, b0_ref,
                    gs0_ref, gb0_ref, w1_ref, b1_ref, gs1_ref, gb1_ref,
                    out_ref):
    it = jnp.concatenate([it0_ref[0], it1_ref[0]], axis=0)
    f2 = f2_ref[0]
    n2 = it.shape[0]
    h = (lax.dot_general(w0a_ref[...], f2, (((1,), (0,)), ((), ())),
                         preferred_element_type=jnp.float32,
                         precision=lax.Precision.HIGHEST)
         + lax.dot_general(w0b_ref[...], it, (((1,), (1,)), ((), ())),
                           preferred_element_type=jnp.float32,
                           precision=lax.Precision.HIGHEST)
         + b0_ref[...])
    s = jnp.sum(h, axis=1, keepdims=True)
    ss = jnp.sum(h * h, axis=1, keepdims=True)
    mean, var = _group_stats_col(s, ss, 16, n2 * 16)
    h = (h - mean) * lax.rsqrt(var + 1e-5) * gs0_ref[...] + gb0_ref[...]
    h = jnp.where(h >= 0, h, 0.1 * h)
    h2 = lax.dot_general(w1_ref[...], h, (((1,), (0,)), ((), ())),
                         preferred_element_type=jnp.float32,
                         precision=lax.Precision.HIGHEST) + b1_ref[...]
    s2 = jnp.sum(h2, axis=1, keepdims=True)
    ss2 = jnp.sum(h2 * h2, axis=1, keepdims=True)
    mean2, var2 = _group_stats_col(s2, ss2, 16, n2 * 16)
    h2 = (h2 - mean2) * lax.rsqrt(var2 + 1e-5) * gs1_ref[...] + gb1_ref[...]
    out_ref[0] = jnp.where(h2 >= 0, h2, 0.1 * h2)


def _mlp(interp0, interp1, feat2, w0a, w0b, b0, gs0, gb0, w1, b1, gs1, gb1):
    B, nh, co = interp0.shape
    N2 = 2 * nh
    c2 = feat2.shape[1]
    full = lambda shape: pl.BlockSpec(shape, lambda b: tuple(0 for _ in shape))
    return pl.pallas_call(
        _mlp_split_body,
        grid=(B,),
        in_specs=[
            pl.BlockSpec((1, nh, co), lambda b: (b, 0, 0)),
            pl.BlockSpec((1, nh, co), lambda b: (b, 0, 0)),
            pl.BlockSpec((1, c2, N2), lambda b: (b, 0, 0)),
            full(w0a.shape), full(w0b.shape), full(b0.shape),
            full(gs0.shape), full(gb0.shape), full(w1.shape),
            full(b1.shape), full(gs1.shape), full(gb1.shape),
        ],
        out_specs=pl.BlockSpec((1, co, N2), lambda b: (b, 0, 0)),
        out_shape=jax.ShapeDtypeStruct((B, co, N2), jnp.float32),
    )(interp0, interp1, feat2, w0a, w0b, b0, gs0, gb0, w1, b1, gs1, gb1)


def kernel(xyz1, xyz2, feat1, feat2, W0, b0, gs0, gb0, W1, b1, gs1, gb1):
    B, _, N1 = xyz1.shape
    N2 = xyz2.shape[2]
    C1 = feat1.shape[1]
    C2 = feat2.shape[1]

    xyz2_t = jnp.transpose(xyz2, (0, 2, 1))
    table = jnp.transpose(feat1, (0, 2, 1))
    table = jnp.concatenate(
        [table, jnp.zeros_like(table)], axis=-1).reshape(B * N1, 2 * C1)

    nh = N2 // 2
    interps = []
    for idx3, w3 in [_knn(xyz1, xyz2_t[:, :nh]), _knn(xyz1, xyz2_t[:, nh:])]:
        idx_flat = idx3.reshape(-1)
        w_km = w3.reshape(B * nh, K_NN).T
        interps.append(
            _sc_interp(idx_flat, w_km, table, C1).reshape(B, nh, C1))

    w0a = W0[:, :C2]
    w0b = W0[:, C2:]
    return _mlp(interps[0], interps[1], feat2, w0a, w0b,
                b0.reshape(-1, 1), gs0.reshape(-1, 1), gb0.reshape(-1, 1),
                W1, b1.reshape(-1, 1), gs1.reshape(-1, 1),
                gb1.reshape(-1, 1))
